# Initial kernel scaffold; baseline (speedup 1.0000x reference)
import jax
import jax.numpy as jnp
from jax import lax
from jax.experimental import pallas as pl
from jax.experimental.pallas import tpu as pltpu
from jax.experimental.pallas import tpu_sc as plsc

_BATCH = 16384
_SEQ = 200
_D = 32
_N = _BATCH * _SEQ
_NC = 2
_NS = 16
_L = 16
_NW = _NC * _NS
_PW = _N // _NW
_C = 1024
_G = _PW // _C
_SUB = 128
_NSUB = _C // _SUB


def _embed_body(tags_hbm, table_hbm, out_hbm, idx_v, rows_v, gsem):
    wid = lax.axis_index("s") * _NC + lax.axis_index("c")
    base = wid * _PW

    def chunk(g, carry):
        off = base + g * _C
        pltpu.sync_copy(tags_hbm.at[pl.ds(off, _C)], idx_v)

        def add1(j, c):
            sl = pl.ds(j * _L, _L)
            idx_v[sl] = idx_v[sl] + 1
            return c

        lax.fori_loop(0, _C // _L, add1, 0, unroll=True)

        copies = []
        for i in range(_NSUB):
            copies.append(pltpu.async_copy(
                table_hbm.at[idx_v.at[pl.ds(i * _SUB, _SUB)]],
                rows_v.at[pl.ds(i * _SUB, _SUB), :],
                gsem,
            ))
        for c in copies:
            c.wait()
        pltpu.sync_copy(rows_v, out_hbm.at[pl.ds(off, _C), :])
        return carry

    lax.fori_loop(0, _G, chunk, 0)


@jax.jit
def kernel(tags, table):
    tags_flat = tags.reshape(_N)
    mesh = plsc.VectorSubcoreMesh(
        core_axis_name="c", subcore_axis_name="s",
        num_cores=_NC, num_subcores=_NS,
    )
    out = pl.kernel(
        _embed_body,
        out_type=jax.ShapeDtypeStruct((_N, _D), jnp.float32),
        mesh=mesh,
        scratch_types=[
            pltpu.VMEM((_C,), jnp.int32),
            pltpu.VMEM((_C, _D), jnp.float32),
            pltpu.SemaphoreType.DMA,
        ],
    )(tags_flat, table)
    return out.reshape(_BATCH, _SEQ, _D)


# speedup vs baseline: 4.7939x; 4.7939x over previous
import jax
import jax.numpy as jnp
from jax import lax
from jax.experimental import pallas as pl
from jax.experimental.pallas import tpu as pltpu
from jax.experimental.pallas import tpu_sc as plsc

_BATCH = 16384
_SEQ = 200
_D = 32
_N = _BATCH * _SEQ
_NC = 2
_NS = 16
_L = 16
_NW = _NC * _NS
_PW = _N // _NW
_C = 1024
_G = _PW // _C
_SUB = 128
_NSUB = _C // _SUB


def _embed_body(tags_hbm, table_hbm, out_hbm, idx_v, rows_v, gsem):
    wid = lax.axis_index("s") * _NC + lax.axis_index("c")
    base = wid * _PW

    def chunk(g, carry):
        off = base + g * _C
        pltpu.sync_copy(tags_hbm.at[pl.ds(off, _C)], idx_v)

        def add1(j, c):
            sl = pl.ds(j * _L, _L)
            idx_v[sl] = idx_v[sl] + 1
            return c

        lax.fori_loop(0, _C // _L, add1, 0, unroll=True)

        copies = []
        for i in range(_NSUB):
            copies.append(pltpu.async_copy(
                table_hbm.at[idx_v.at[pl.ds(i * _SUB, _SUB)]],
                rows_v.at[pl.ds(i * _SUB, _SUB), :],
                gsem,
            ))
        for c in copies:
            c.wait()
        pltpu.sync_copy(rows_v, out_hbm.at[pl.ds(off, _C), :])
        return carry

    lax.fori_loop(0, _G, chunk, 0)


@jax.jit
def kernel(tags, table):
    tags_flat = tags.reshape(_N)
    mesh = plsc.VectorSubcoreMesh(
        core_axis_name="c", subcore_axis_name="s",
        num_cores=_NC, num_subcores=_NS,
    )
    out = pl.kernel(
        _embed_body,
        out_type=jax.ShapeDtypeStruct((_N, _D), jnp.float32),
        mesh=mesh,
        scratch_types=[
            pltpu.VMEM((_C,), jnp.int32),
            pltpu.VMEM((_C, _D), jnp.float32),
            pltpu.SemaphoreType.DMA,
        ],
        compiler_params=pltpu.CompilerParams(use_tc_tiling_on_sc=False),
    )(tags_flat, table)
    return out.reshape(_BATCH, _SEQ, _D)


# speedup vs baseline: 4.9477x; 1.0321x over previous
import jax
import jax.numpy as jnp
from jax import lax
from jax.experimental import pallas as pl
from jax.experimental.pallas import tpu as pltpu
from jax.experimental.pallas import tpu_sc as plsc

_BATCH = 16384
_SEQ = 200
_D = 32
_N = _BATCH * _SEQ
_NC = 2
_NS = 16
_L = 16
_NW = _NC * _NS
_PW = _N // _NW
_C = 1024
_G = _PW // _C
_SUB = 128
_NSUB = _C // _SUB


def _embed_body(tags_hbm, table_hbm, out_hbm,
                idx0, idx1, rows0, rows1,
                gsem0, gsem1, osem0, osem1):
    idx = (idx0, idx1)
    rows = (rows0, rows1)
    gsem = (gsem0, gsem1)
    osem = (osem0, osem1)

    wid = lax.axis_index("s") * _NC + lax.axis_index("c")
    base = wid * _PW

    def load_idx(b, g):
        pltpu.sync_copy(tags_hbm.at[pl.ds(base + g * _C, _C)], idx[b])
        for j in range(_C // _L):
            sl = pl.ds(j * _L, _L)
            idx[b][sl] = idx[b][sl] + 1

    def fire_gathers(b):
        for i in range(_NSUB):
            pltpu.async_copy(
                table_hbm.at[idx[b].at[pl.ds(i * _SUB, _SUB)]],
                rows[b].at[pl.ds(i * _SUB, _SUB), :],
                gsem[b],
            )

    def wait_gathers(b):
        pltpu.make_async_copy(table_hbm.at[idx[b]], rows[b], gsem[b]).wait()

    def fire_out(b, g):
        pltpu.async_copy(rows[b], out_hbm.at[pl.ds(base + g * _C, _C), :],
                         osem[b])

    def wait_out(b, g):
        pltpu.make_async_copy(rows[b], out_hbm.at[pl.ds(base + g * _C, _C), :],
                              osem[b]).wait()

    load_idx(0, 0)
    fire_gathers(0)
    load_idx(1, 1)
    wait_gathers(0)
    fire_out(0, 0)
    load_idx(0, 2)
    fire_gathers(1)
    wait_gathers(1)
    fire_out(1, 1)
    load_idx(1, 3)
    wait_out(0, 0)
    fire_gathers(0)

    def pair(k, carry):
        g0 = 2 * k
        wait_gathers(0)
        fire_out(0, g0)
        load_idx(0, g0 + 2)
        wait_out(1, g0 - 1)
        fire_gathers(1)
        wait_gathers(1)
        fire_out(1, g0 + 1)
        load_idx(1, g0 + 3)
        wait_out(0, g0)
        fire_gathers(0)
        return carry

    lax.fori_loop(1, _G // 2 - 1, pair, 0)

    wait_gathers(0)
    fire_out(0, _G - 2)
    wait_out(1, _G - 3)
    fire_gathers(1)
    wait_gathers(1)
    fire_out(1, _G - 1)
    wait_out(0, _G - 2)
    wait_out(1, _G - 1)


@jax.jit
def kernel(tags, table):
    tags_flat = tags.reshape(_N)
    mesh = plsc.VectorSubcoreMesh(
        core_axis_name="c", subcore_axis_name="s",
        num_cores=_NC, num_subcores=_NS,
    )
    out = pl.kernel(
        _embed_body,
        out_type=jax.ShapeDtypeStruct((_N, _D), jnp.float32),
        mesh=mesh,
        scratch_types=[
            pltpu.VMEM((_C,), jnp.int32),
            pltpu.VMEM((_C,), jnp.int32),
            pltpu.VMEM((_C, _D), jnp.float32),
            pltpu.VMEM((_C, _D), jnp.float32),
            pltpu.SemaphoreType.DMA,
            pltpu.SemaphoreType.DMA,
            pltpu.SemaphoreType.DMA,
            pltpu.SemaphoreType.DMA,
        ],
        compiler_params=pltpu.CompilerParams(use_tc_tiling_on_sc=False),
    )(tags_flat, table)
    return out.reshape(_BATCH, _SEQ, _D)


# speedup vs baseline: 4.9983x; 1.0102x over previous
import jax
import jax.numpy as jnp
from jax import lax
from jax.experimental import pallas as pl
from jax.experimental.pallas import tpu as pltpu
from jax.experimental.pallas import tpu_sc as plsc

_BATCH = 16384
_SEQ = 200
_D = 32
_N = _BATCH * _SEQ
_NC = 2
_NS = 16
_L = 16
_NW = _NC * _NS
_RW = _BATCH // _NW
_CR = 8
_C = _CR * _SEQ
_G = _RW // _CR
_S0 = 128
_S1 = _SEQ - _S0


def _embed_body(tags_hbm, table_hbm, out_hbm,
                idx0, idx1, rows0, rows1,
                gsem0, gsem1, osem0, osem1):
    idx = (idx0, idx1)
    rows = (rows0, rows1)
    gsem = (gsem0, gsem1)
    osem = (osem0, osem1)

    wid = lax.axis_index("s") * _NC + lax.axis_index("c")
    base = wid * _RW * _SEQ
    rbase = wid * _RW

    def load_idx(b, g):
        pltpu.sync_copy(tags_hbm.at[pl.ds(base + g * _C, _C)], idx[b])
        for j in range(_C // _L):
            sl = pl.ds(j * _L, _L)
            idx[b][sl] = idx[b][sl] + 1

    def fire_gathers(b):
        for i in range(_CR):
            pltpu.async_copy(
                table_hbm.at[idx[b].at[pl.ds(i * _SEQ, _S0)]],
                rows[b].at[i, pl.ds(0, _S0), :],
                gsem[b],
            )
            pltpu.async_copy(
                table_hbm.at[idx[b].at[pl.ds(i * _SEQ + _S0, _S1)]],
                rows[b].at[i, pl.ds(_S0, _S1), :],
                gsem[b],
            )

    def wait_gathers(b):
        pltpu.make_async_copy(out_hbm.at[pl.ds(rbase, _CR), :, :],
                              rows[b], gsem[b]).wait()

    def fire_out(b, g):
        pltpu.async_copy(rows[b], out_hbm.at[pl.ds(rbase + g * _CR, _CR), :, :],
                         osem[b])

    def wait_out(b, g):
        pltpu.make_async_copy(rows[b],
                              out_hbm.at[pl.ds(rbase + g * _CR, _CR), :, :],
                              osem[b]).wait()

    load_idx(0, 0)
    fire_gathers(0)
    load_idx(1, 1)
    wait_gathers(0)
    fire_out(0, 0)
    load_idx(0, 2)
    fire_gathers(1)
    wait_gathers(1)
    fire_out(1, 1)
    load_idx(1, 3)
    wait_out(0, 0)
    fire_gathers(0)

    def pair(k, carry):
        g0 = 2 * k
        wait_gathers(0)
        fire_out(0, g0)
        load_idx(0, g0 + 2)
        wait_out(1, g0 - 1)
        fire_gathers(1)
        wait_gathers(1)
        fire_out(1, g0 + 1)
        load_idx(1, g0 + 3)
        wait_out(0, g0)
        fire_gathers(0)
        return carry

    lax.fori_loop(1, _G // 2 - 1, pair, 0)

    wait_gathers(0)
    fire_out(0, _G - 2)
    wait_out(1, _G - 3)
    fire_gathers(1)
    wait_gathers(1)
    fire_out(1, _G - 1)
    wait_out(0, _G - 2)
    wait_out(1, _G - 1)


@jax.jit
def kernel(tags, table):
    tags_flat = tags.reshape(_N)
    mesh = plsc.VectorSubcoreMesh(
        core_axis_name="c", subcore_axis_name="s",
        num_cores=_NC, num_subcores=_NS,
    )
    return pl.kernel(
        _embed_body,
        out_type=jax.ShapeDtypeStruct((_BATCH, _SEQ, _D), jnp.float32),
        mesh=mesh,
        scratch_types=[
            pltpu.VMEM((_C,), jnp.int32),
            pltpu.VMEM((_C,), jnp.int32),
            pltpu.VMEM((_CR, _SEQ, _D), jnp.float32),
            pltpu.VMEM((_CR, _SEQ, _D), jnp.float32),
            pltpu.SemaphoreType.DMA,
            pltpu.SemaphoreType.DMA,
            pltpu.SemaphoreType.DMA,
            pltpu.SemaphoreType.DMA,
        ],
        compiler_params=pltpu.CompilerParams(use_tc_tiling_on_sc=False),
    )(tags_flat, table)
